# Initial kernel scaffold; baseline (speedup 1.0000x reference)
import jax
import jax.numpy as jnp
from jax.experimental import pallas as pl


def kernel(user_emb, item_emb, edge_index, edge_values):
    raise NotImplementedError("write your pallas kernel here")


# speedup vs baseline: 2.7442x; 2.7442x over previous
import functools

import jax
import jax.numpy as jnp
from jax import lax
from jax.experimental import pallas as pl
from jax.experimental.pallas import tpu as pltpu
from jax.experimental.pallas import tpu_sc as plsc

_N_USERS = 25000
_N_NODES = 50000
_D = 64
_E = 800000

_NC = 2
_NS = 16
_CHUNK = 128
_E_PAD = 802816
_CH_PER_SC = _E_PAD // _CHUNK
_CH_PER_TILE = _CH_PER_SC // _NS
_HALF = _N_NODES // _NC
_ACC_ROWS = _HALF + 88
_ZR = 32
_CP_ROWS = 1560


def _prop_body(table, rows, cols, vals, out,
               cidx, rraw, rloc, vbuf, gbuf, zbuf, acc, sem):
    core = lax.axis_index("c")
    sid = lax.axis_index("s")
    base_row = core * _HALF

    zero = jnp.zeros((16,), jnp.float32)
    for r in range(_ZR):
        for j in range(_D // 16):
            zbuf[r, pl.ds(j * 16, 16)] = zero

    def zloop(i, carry):
        pltpu.sync_copy(zbuf, acc.at[pl.ds(sid * 1568 + i * _ZR, _ZR)])
        return carry

    lax.fori_loop(0, 1568 // _ZR, zloop, 0)
    plsc.subcore_barrier()

    def chunk_body(c, carry):
        ebase = (sid * _CH_PER_TILE + c) * _CHUNK
        pltpu.sync_copy(cols.at[pl.ds(ebase, _CHUNK)], cidx)
        pltpu.sync_copy(rows.at[pl.ds(ebase, _CHUNK)], rraw)
        pltpu.sync_copy(vals.at[pl.ds(ebase, _CHUNK)], vbuf)
        pltpu.async_copy(table.at[cidx], gbuf, sem).wait()
        for g in range(_CHUNK // 16):
            r = rraw[pl.ds(g * 16, 16)]
            loc = r - base_row
            oob = (loc < 0) | (loc >= _HALF)
            rloc[pl.ds(g * 16, 16)] = jnp.where(oob, _HALF, loc)
        for g in range(_CHUNK // 16):
            vv = vbuf[pl.ds(g * 16, 16)]
            for k in range(16):
                i = g * 16 + k
                v = vv[k]
                for j in range(_D // 16):
                    sl = pl.ds(j * 16, 16)
                    gbuf[i, sl] = gbuf[i, sl] * v
        pltpu.sync_copy(gbuf, acc.at[rloc], add=True)
        return carry

    lax.fori_loop(0, _CH_PER_TILE, chunk_body, 0)
    plsc.subcore_barrier()

    pltpu.sync_copy(acc.at[pl.ds(sid * _CP_ROWS, _CP_ROWS)],
                    out.at[pl.ds(base_row + sid * _CP_ROWS, _CP_ROWS)])

    @pl.when(sid < 5)
    def _():
        tail = _NS * _CP_ROWS + sid * 8
        pltpu.sync_copy(acc.at[pl.ds(tail, 8)],
                        out.at[pl.ds(base_row + tail, 8)])


_prop = functools.partial(
    pl.kernel,
    mesh=plsc.VectorSubcoreMesh(core_axis_name="c", subcore_axis_name="s"),
    compiler_params=pltpu.CompilerParams(use_tc_tiling_on_sc=False),
    out_type=jax.ShapeDtypeStruct((_N_NODES, _D), jnp.float32),
    scratch_types=[
        pltpu.VMEM((_CHUNK,), jnp.int32),
        pltpu.VMEM((_CHUNK,), jnp.int32),
        pltpu.VMEM((_CHUNK,), jnp.int32),
        pltpu.VMEM((_CHUNK,), jnp.float32),
        pltpu.VMEM((_CHUNK, _D), jnp.float32),
        pltpu.VMEM((_ZR, _D), jnp.float32),
        pltpu.VMEM_SHARED((_ACC_ROWS, _D), jnp.float32),
        pltpu.SemaphoreType.DMA,
    ],
)(_prop_body)


def _mean_body(a, b, c, d, o):
    o[...] = (a[...] + b[...] + c[...] + d[...]) * 0.25


def _mean(x0, x1, x2, x3):
    blk = (1000, _D)
    spec = pl.BlockSpec(blk, lambda i: (i, 0))
    return pl.pallas_call(
        _mean_body,
        grid=(_N_NODES // blk[0],),
        in_specs=[spec] * 4,
        out_specs=spec,
        out_shape=jax.ShapeDtypeStruct((_N_NODES, _D), jnp.float32),
    )(x0, x1, x2, x3)


def kernel(user_emb, item_emb, edge_index, edge_values):
    rows = jnp.asarray(edge_index[0], jnp.int32)
    cols = jnp.asarray(edge_index[1], jnp.int32)
    vals = edge_values.astype(jnp.float32)
    pad = _E_PAD - _E
    rows = jnp.concatenate([rows, jnp.zeros((pad,), jnp.int32)])
    cols = jnp.concatenate([cols, jnp.zeros((pad,), jnp.int32)])
    vals = jnp.concatenate([vals, jnp.zeros((pad,), jnp.float32)])

    x0 = jnp.concatenate([user_emb, item_emb], axis=0)
    x1 = _prop(x0, rows, cols, vals)
    x2 = _prop(x1, rows, cols, vals)
    x3 = _prop(x2, rows, cols, vals)
    m = _mean(x0, x1, x2, x3)
    return m[:_N_USERS], m[_N_USERS:]


# speedup vs baseline: 4.9970x; 1.8209x over previous
import functools

import jax
import jax.numpy as jnp
from jax import lax
from jax.experimental import pallas as pl
from jax.experimental.pallas import tpu as pltpu
from jax.experimental.pallas import tpu_sc as plsc

_N_USERS = 25000
_N_NODES = 50000
_D = 64
_E = 800000

_NC = 2
_NS = 16
_CHUNK = 128
_E_PAD = 802816
_NCH = _E_PAD // _CHUNK
_CH_PER_TILE = _NCH // _NS
_HALF = _N_NODES // _NC
_ACC_ROWS = _HALF + 88
_ZR = 32
_CP_ROWS = 1560


def _scale_chunk(ebuf, vbuf, gbuf, rloc, base_row):
    for g in range(_CHUNK // 16):
        sl16 = pl.ds(g * 16, 16)
        r = ebuf[1, sl16]
        loc = r - base_row
        oob = (loc < 0) | (loc >= _HALF)
        rloc[sl16] = jnp.where(oob, _HALF, loc)
        vv = vbuf[sl16]
        for k in range(16):
            i = g * 16 + k
            v = vv[k]
            for j in range(_D // 16):
                sl = pl.ds(j * 16, 16)
                gbuf[i, sl] = gbuf[i, sl] * v


def _prop_body(table, packed, valsh, out,
               ebuf0, ebuf1, vbuf0, vbuf1, gbuf0, gbuf1, rloc0, rloc1,
               zbuf, acc, se0, se1, sg0, sg1, ss0, ss1):
    core = lax.axis_index("c")
    sid = lax.axis_index("s")
    base_row = core * _HALF
    ebuf, vbuf = (ebuf0, ebuf1), (vbuf0, vbuf1)
    gbuf, rloc = (gbuf0, gbuf1), (rloc0, rloc1)
    se, sg, ss = (se0, se1), (sg0, sg1), (ss0, ss1)

    q0 = sid * _CH_PER_TILE

    def eload(c, b):
        qc = jnp.minimum(q0 + c, _NCH - 1)
        pltpu.async_copy(packed.at[qc], ebuf[b], se[b])
        pltpu.async_copy(valsh.at[pl.ds(qc * _CHUNK, _CHUNK)], vbuf[b], se[b])

    def ewait(b):
        pltpu.make_async_copy(packed.at[q0], ebuf[b], se[b]).wait()
        pltpu.make_async_copy(valsh.at[pl.ds(0, _CHUNK)], vbuf[b], se[b]).wait()

    def gather(b):
        pltpu.async_copy(table.at[ebuf[b].at[0]], gbuf[b], sg[b])

    zero = jnp.zeros((16,), jnp.float32)
    for r in range(_ZR):
        for j in range(_D // 16):
            zbuf[r, pl.ds(j * 16, 16)] = zero

    def zloop(i, carry):
        pltpu.sync_copy(zbuf, acc.at[pl.ds(sid * 1568 + i * _ZR, _ZR)])
        return carry

    lax.fori_loop(0, 1568 // _ZR, zloop, 0)

    eload(0, 0)
    ewait(0)
    gather(0)
    eload(1, 1)
    plsc.subcore_barrier()

    def chunk_pair(i, carry):
        for b in (0, 1):
            c = 2 * i + b
            nb = 1 - b
            ewait(nb)

            @pl.when(c >= 1)
            def _():
                pltpu.make_async_copy(gbuf[nb], acc.at[rloc[nb]], ss[nb]).wait()

            gather(nb)
            pltpu.make_async_copy(table.at[ebuf[b].at[0]], gbuf[b], sg[b]).wait()
            _scale_chunk(ebuf[b], vbuf[b], gbuf[b], rloc[b], base_row)
            pltpu.async_copy(gbuf[b], acc.at[rloc[b]], ss[b], add=True)
            eload(c + 2, b)
        return carry

    lax.fori_loop(0, _CH_PER_TILE // 2, chunk_pair, 0)

    pltpu.make_async_copy(gbuf[1], acc.at[rloc[1]], ss[1]).wait()
    pltpu.make_async_copy(table.at[ebuf[0].at[0]], gbuf[0], sg[0]).wait()
    ewait(1)
    plsc.subcore_barrier()

    pltpu.sync_copy(acc.at[pl.ds(sid * _CP_ROWS, _CP_ROWS)],
                    out.at[pl.ds(base_row + sid * _CP_ROWS, _CP_ROWS)])

    @pl.when(sid < 5)
    def _():
        tail = _NS * _CP_ROWS + sid * 8
        pltpu.sync_copy(acc.at[pl.ds(tail, 8)],
                        out.at[pl.ds(base_row + tail, 8)])


_prop = functools.partial(
    pl.kernel,
    mesh=plsc.VectorSubcoreMesh(core_axis_name="c", subcore_axis_name="s"),
    compiler_params=pltpu.CompilerParams(use_tc_tiling_on_sc=False),
    out_type=jax.ShapeDtypeStruct((_N_NODES, _D), jnp.float32),
    scratch_types=[
        pltpu.VMEM((2, _CHUNK), jnp.int32),
        pltpu.VMEM((2, _CHUNK), jnp.int32),
        pltpu.VMEM((_CHUNK,), jnp.float32),
        pltpu.VMEM((_CHUNK,), jnp.float32),
        pltpu.VMEM((_CHUNK, _D), jnp.float32),
        pltpu.VMEM((_CHUNK, _D), jnp.float32),
        pltpu.VMEM((_CHUNK,), jnp.int32),
        pltpu.VMEM((_CHUNK,), jnp.int32),
        pltpu.VMEM((_ZR, _D), jnp.float32),
        pltpu.VMEM_SHARED((_ACC_ROWS, _D), jnp.float32),
        pltpu.SemaphoreType.DMA,
        pltpu.SemaphoreType.DMA,
        pltpu.SemaphoreType.DMA,
        pltpu.SemaphoreType.DMA,
        pltpu.SemaphoreType.DMA,
        pltpu.SemaphoreType.DMA,
    ],
)(_prop_body)


def _mean_body(a, b, c, d, o):
    o[...] = (a[...] + b[...] + c[...] + d[...]) * 0.25


def _mean(x0, x1, x2, x3):
    blk = (1000, _D)
    spec = pl.BlockSpec(blk, lambda i: (i, 0))
    return pl.pallas_call(
        _mean_body,
        grid=(_N_NODES // blk[0],),
        in_specs=[spec] * 4,
        out_specs=spec,
        out_shape=jax.ShapeDtypeStruct((_N_NODES, _D), jnp.float32),
    )(x0, x1, x2, x3)


def kernel(user_emb, item_emb, edge_index, edge_values):
    rows = jnp.asarray(edge_index[0], jnp.int32)
    cols = jnp.asarray(edge_index[1], jnp.int32)
    vals = edge_values.astype(jnp.float32)
    pad = _E_PAD - _E
    rows = jnp.concatenate([rows, jnp.zeros((pad,), jnp.int32)])
    cols = jnp.concatenate([cols, jnp.zeros((pad,), jnp.int32)])
    vals = jnp.concatenate([vals, jnp.zeros((pad,), jnp.float32)])
    packed = jnp.stack([cols.reshape(_NCH, _CHUNK),
                        rows.reshape(_NCH, _CHUNK)], axis=1)

    x0 = jnp.concatenate([user_emb, item_emb], axis=0)
    x1 = _prop(x0, packed, vals)
    x2 = _prop(x1, packed, vals)
    x3 = _prop(x2, packed, vals)
    m = _mean(x0, x1, x2, x3)
    return m[:_N_USERS], m[_N_USERS:]
